# baseline (speedup 1.0000x reference)
import jax
import jax.numpy as jnp
from jax.experimental import pallas as pl
from jax.experimental.pallas import tpu as pltpu

_B, _S, _DIN = 4, 512, 512
_K, _D, _N = 8192, 32, 32
_T = _B * _S
_TT = 512
_KC = 2048
_NSEM = 10
_BF = jnp.bfloat16
_F32 = jnp.float32


def _bf16_matmul(a, b, dims):
    return jax.lax.dot_general(
        a.astype(_BF), b.astype(_BF), dimension_numbers=(dims, ((), ())),
        preferred_element_type=_F32)


def _rvq_kernel(audioT_ref, semT_ref, win_ref, binc_ref, cb_ref, cbT_ref,
                wsem_ref, bsemc_ref, quantT_ref, idx_ref, loss_ref, xT_ref,
                resT_ref):
    i = pl.program_id(0)

    @pl.when(i == 0)
    def _init():
        xT = _bf16_matmul(win_ref[...], audioT_ref[...], ((0,), (0,)))
        xT = xT + binc_ref[...]
        xT_ref[...] = xT
        resT_ref[...] = xT
        quantT_ref[...] = jnp.zeros_like(xT)
        loss_ref[0, 0] = 0.0

    @pl.when(i < _NSEM)
    def _bias():
        biasT = _bf16_matmul(wsem_ref[i], semT_ref[...], ((0,), (0,)))
        biasT = biasT + bsemc_ref[i]
        for b in range(_B):
            resT_ref[:, b * _S:(b + 1) * _S] = (
                resT_ref[:, b * _S:(b + 1) * _S] + 0.1 * biasT[:, b:b + 1])

    cb = cb_ref[0]
    b2 = jnp.sum(cb * cb, axis=1, keepdims=True)
    cb_bf = cb.astype(_BF)
    cbT = cbT_ref[0]

    kiota = jax.lax.broadcasted_iota(jnp.int32, (_KC, _TT), 0)

    def _tree8(val, ix):
        while val.shape[0] > 8:
            h = val.shape[0] // 2
            take_b = val[h:] < val[:h]
            val = jnp.where(take_b, val[h:], val[:h])
            ix = jnp.where(take_b, ix[h:], ix[:h])
        return val, ix

    def _argmin(sl):
        r = resT_ref[:, sl]
        a2 = jnp.sum(r * r, axis=0, keepdims=True)
        val8, ix8 = None, None
        for c in range(_K // _KC):
            ab = _bf16_matmul(cb_bf[c * _KC:(c + 1) * _KC], r, ((1,), (0,)))
            d2 = (a2 - 2.0 * ab) + b2[c * _KC:(c + 1) * _KC]
            v, ik = _tree8(d2, kiota + c * _KC)
            if val8 is None:
                val8, ix8 = v, ik
            else:
                tb = v < val8
                val8 = jnp.where(tb, v, val8)
                ix8 = jnp.where(tb, ik, ix8)
        m = jnp.min(val8, axis=0, keepdims=True)
        idx = jnp.min(jnp.where(val8 == m, ix8, _K), axis=0,
                      keepdims=True)
        return r, idx

    def _gather_update(sl, r, idx, loss_acc):
        idx_ref[0, 0, sl] = idx[0]
        lob = jnp.broadcast_to(idx & 127, (_D, _TT))
        level = [
            jnp.take_along_axis(cbT[:, 128 * g:128 * (g + 1)], lob, axis=1)
            for g in range(_K // 128)
        ]
        hi = idx >> 7
        bit = 1
        while len(level) > 1:
            mask = (hi & bit) != 0
            level = [jnp.where(mask, level[j + 1], level[j])
                     for j in range(0, len(level), 2)]
            bit <<= 1
        qT = level[0]
        quantT_ref[:, sl] = quantT_ref[:, sl] + qT
        resT_ref[:, sl] = r - qT
        diff = qT - xT_ref[:, sl]
        return loss_acc + jnp.sum(diff * diff)

    def _tile(u, loss_acc):
        sl = pl.ds(u * _TT, _TT)
        r, idx = _argmin(sl)
        return _gather_update(sl, r, idx, loss_acc)

    loss_i = jax.lax.fori_loop(0, _T // _TT, _tile, jnp.float32(0.0))
    loss_ref[0, 0] = loss_ref[0, 0] + loss_i / (_T * _D)

    @pl.when(i == _N - 1)
    def _finish():
        xT = xT_ref[...]
        quantT_ref[...] = xT + (quantT_ref[...] - xT)


@jax.jit
def kernel(audio_features, semantic_context, W_in, b_in, codebooks, W_sem,
           b_sem):
    audioT = jnp.reshape(audio_features, (_T, _DIN)).T
    semT = semantic_context.T
    binc = jnp.reshape(b_in, (_D, 1))
    bsemc = jnp.reshape(b_sem, (_NSEM, _D, 1))
    codebooksT = jnp.transpose(codebooks, (0, 2, 1))

    quantT, idx, loss = pl.pallas_call(
        _rvq_kernel,
        grid=(_N,),
        in_specs=[
            pl.BlockSpec((_DIN, _T), lambda i: (0, 0)),
            pl.BlockSpec((4096, _B), lambda i: (0, 0)),
            pl.BlockSpec((_DIN, _D), lambda i: (0, 0)),
            pl.BlockSpec((_D, 1), lambda i: (0, 0)),
            pl.BlockSpec((1, _K, _D), lambda i: (i, 0, 0)),
            pl.BlockSpec((1, _D, _K), lambda i: (i, 0, 0)),
            pl.BlockSpec((_NSEM, 4096, _D), lambda i: (0, 0, 0)),
            pl.BlockSpec((_NSEM, _D, 1), lambda i: (0, 0, 0)),
        ],
        out_specs=[
            pl.BlockSpec((_D, _T), lambda i: (0, 0)),
            pl.BlockSpec((1, 1, _T), lambda i: (i, 0, 0)),
            pl.BlockSpec(memory_space=pltpu.SMEM),
        ],
        out_shape=[
            jax.ShapeDtypeStruct((_D, _T), _F32),
            jax.ShapeDtypeStruct((_N, 1, _T), jnp.int32),
            jax.ShapeDtypeStruct((1, 1), _F32),
        ],
        scratch_shapes=[
            pltpu.VMEM((_D, _T), _F32),
            pltpu.VMEM((_D, _T), _F32),
        ],
        compiler_params=pltpu.CompilerParams(
            dimension_semantics=("arbitrary",)),
    )(audioT, semT, W_in, binc, codebooks, codebooksT, W_sem, bsemc)

    quantized = jnp.reshape(quantT.T, (_B, _S, _D))
    indices = jnp.reshape(idx, (_N, _B, _S))
    commitment_loss = jnp.reshape(loss, ()) * 0.25
    return (quantized, indices, commitment_loss)


# speedup vs baseline: 1.1052x; 1.1052x over previous
import jax
import jax.numpy as jnp
from jax.experimental import pallas as pl
from jax.experimental.pallas import tpu as pltpu

_B, _S, _DIN = 4, 512, 512
_K, _D, _N = 8192, 32, 32
_T = _B * _S
_TT = 256
_KC = 2048
_NSEM = 10
_BF = jnp.bfloat16
_F32 = jnp.float32


def _bf16_matmul(a, b, dims):
    return jax.lax.dot_general(
        a.astype(_BF), b.astype(_BF), dimension_numbers=(dims, ((), ())),
        preferred_element_type=_F32)


def _rvq_kernel(audioT_ref, semT_ref, win_ref, binc_ref, cb_ref, cbT_ref,
                wsem_ref, bsemc_ref, quantT_ref, idx_ref, loss_ref, xT_ref,
                resT_ref):
    i = pl.program_id(0)

    @pl.when(i == 0)
    def _init():
        xT = _bf16_matmul(win_ref[...], audioT_ref[...], ((0,), (0,)))
        xT = xT + binc_ref[...]
        xT_ref[...] = xT
        resT_ref[...] = xT
        quantT_ref[...] = jnp.zeros_like(xT)
        loss_ref[0, 0] = 0.0

    @pl.when(i < _NSEM)
    def _bias():
        biasT = _bf16_matmul(wsem_ref[i], semT_ref[...], ((0,), (0,)))
        biasT = biasT + bsemc_ref[i]
        for b in range(_B):
            resT_ref[:, b * _S:(b + 1) * _S] = (
                resT_ref[:, b * _S:(b + 1) * _S] + 0.1 * biasT[:, b:b + 1])

    cb = cb_ref[0]
    b2 = jnp.sum(cb * cb, axis=1, keepdims=True)
    cb_bf = cb.astype(_BF)
    cbT = cbT_ref[0]

    kiota = jax.lax.broadcasted_iota(jnp.int32, (_KC, _TT), 0)

    def _tree8(val, ix):
        while val.shape[0] > 8:
            h = val.shape[0] // 2
            take_b = val[h:] < val[:h]
            val = jnp.where(take_b, val[h:], val[:h])
            ix = jnp.where(take_b, ix[h:], ix[:h])
        return val, ix

    def _argmin(sl):
        r = resT_ref[:, sl]
        a2 = jnp.sum(r * r, axis=0, keepdims=True)
        val8, ix8 = None, None
        for c in range(_K // _KC):
            ab = _bf16_matmul(cb_bf[c * _KC:(c + 1) * _KC], r, ((1,), (0,)))
            d2 = (a2 - 2.0 * ab) + b2[c * _KC:(c + 1) * _KC]
            v, ik = _tree8(d2, kiota + c * _KC)
            if val8 is None:
                val8, ix8 = v, ik
            else:
                tb = v < val8
                val8 = jnp.where(tb, v, val8)
                ix8 = jnp.where(tb, ik, ix8)
        m = jnp.min(val8, axis=0, keepdims=True)
        idx = jnp.min(jnp.where(val8 == m, ix8, _K), axis=0,
                      keepdims=True)
        return r, idx

    def _gather_update(sl, r, idx, loss_acc):
        idx_ref[0, 0, sl] = idx[0]
        lob = jnp.broadcast_to(idx & 127, (_D, _TT))
        level = [
            jnp.take_along_axis(cbT[:, 128 * g:128 * (g + 1)], lob, axis=1)
            for g in range(_K // 128)
        ]
        hi = idx >> 7
        bit = 1
        while len(level) > 1:
            mask = (hi & bit) != 0
            level = [jnp.where(mask, level[j + 1], level[j])
                     for j in range(0, len(level), 2)]
            bit <<= 1
        qT = level[0]
        quantT_ref[:, sl] = quantT_ref[:, sl] + qT
        resT_ref[:, sl] = r - qT
        diff = qT - xT_ref[:, sl]
        return loss_acc + jnp.sum(diff * diff)

    def _tile2(u, loss_acc):
        sl_a = pl.ds(u * 2 * _TT, _TT)
        sl_b = pl.ds(u * 2 * _TT + _TT, _TT)
        r_a, idx_a = _argmin(sl_a)
        loss_acc = _gather_update(sl_a, r_a, idx_a, loss_acc)
        r_b, idx_b = _argmin(sl_b)
        return _gather_update(sl_b, r_b, idx_b, loss_acc)

    loss_i = jax.lax.fori_loop(0, _T // (2 * _TT), _tile2, jnp.float32(0.0))
    loss_ref[0, 0] = loss_ref[0, 0] + loss_i / (_T * _D)

    @pl.when(i == _N - 1)
    def _finish():
        xT = xT_ref[...]
        quantT_ref[...] = xT + (quantT_ref[...] - xT)


@jax.jit
def kernel(audio_features, semantic_context, W_in, b_in, codebooks, W_sem,
           b_sem):
    audioT = jnp.reshape(audio_features, (_T, _DIN)).T
    semT = semantic_context.T
    binc = jnp.reshape(b_in, (_D, 1))
    bsemc = jnp.reshape(b_sem, (_NSEM, _D, 1))
    codebooksT = jnp.transpose(codebooks, (0, 2, 1))

    quantT, idx, loss = pl.pallas_call(
        _rvq_kernel,
        grid=(_N,),
        in_specs=[
            pl.BlockSpec((_DIN, _T), lambda i: (0, 0)),
            pl.BlockSpec((4096, _B), lambda i: (0, 0)),
            pl.BlockSpec((_DIN, _D), lambda i: (0, 0)),
            pl.BlockSpec((_D, 1), lambda i: (0, 0)),
            pl.BlockSpec((1, _K, _D), lambda i: (i, 0, 0)),
            pl.BlockSpec((1, _D, _K), lambda i: (i, 0, 0)),
            pl.BlockSpec((_NSEM, 4096, _D), lambda i: (0, 0, 0)),
            pl.BlockSpec((_NSEM, _D, 1), lambda i: (0, 0, 0)),
        ],
        out_specs=[
            pl.BlockSpec((_D, _T), lambda i: (0, 0)),
            pl.BlockSpec((1, 1, _T), lambda i: (i, 0, 0)),
            pl.BlockSpec(memory_space=pltpu.SMEM),
        ],
        out_shape=[
            jax.ShapeDtypeStruct((_D, _T), _F32),
            jax.ShapeDtypeStruct((_N, 1, _T), jnp.int32),
            jax.ShapeDtypeStruct((1, 1), _F32),
        ],
        scratch_shapes=[
            pltpu.VMEM((_D, _T), _F32),
            pltpu.VMEM((_D, _T), _F32),
        ],
        compiler_params=pltpu.CompilerParams(
            dimension_semantics=("arbitrary",)),
    )(audioT, semT, W_in, binc, codebooks, codebooksT, W_sem, bsemc)

    quantized = jnp.reshape(quantT.T, (_B, _S, _D))
    indices = jnp.reshape(idx, (_N, _B, _S))
    commitment_loss = jnp.reshape(loss, ()) * 0.25
    return (quantized, indices, commitment_loss)
